# baseline (speedup 1.0000x reference)
import functools

import jax
import jax.numpy as jnp
from jax.experimental import pallas as pl
from jax.experimental.pallas import tpu as pltpu


_BLK = 512


def _trim_body(nblk, m_hbm, v_hbm, x_hbm, mo_hbm, vo_hbm, xo_hbm,
               m_v, v_v, mo_v, vo_v, xb_in, xb_out, zbuf,
               msem, vsem, vosem, bsem, csem, zsem):
    pltpu.make_async_copy(m_hbm, m_v, msem).start()
    pltpu.make_async_copy(v_hbm, v_v, vsem).start()
    pltpu.make_async_copy(m_hbm, m_v, msem).wait()
    counts = jnp.sum((m_v[...] != 0).astype(jnp.int32), axis=1)
    ml = jnp.maximum(jnp.max(counts), 1)
    jb = (ml - 1) // _BLK

    pltpu.make_async_copy(
        x_hbm.at[:, pl.ds(jb * _BLK, _BLK)], xb_in, bsem).start()

    zbuf[...] = jnp.zeros_like(zbuf)

    for d in range(nblk):
        @pl.when(d < jb)
        def _copy(d=d):
            pltpu.make_async_copy(
                x_hbm.at[:, pl.ds(d * _BLK, _BLK)],
                xo_hbm.at[:, pl.ds(d * _BLK, _BLK)], csem).start()

        @pl.when(d > jb)
        def _zero(d=d):
            pltpu.make_async_copy(
                zbuf, xo_hbm.at[:, pl.ds(d * _BLK, _BLK)], zsem).start()

    col = jax.lax.broadcasted_iota(jnp.int32, (1, v_v.shape[1]), 1)
    keep = col < ml
    pltpu.make_async_copy(v_hbm, v_v, vsem).wait()
    vo_v[...] = jnp.where(keep, v_v[...], 0.0)
    mo_v[...] = jnp.where(keep & (m_v[...] != 0), 1, 0).astype(jnp.int32)
    pltpu.make_async_copy(vo_v, vo_hbm, vosem).start()
    pltpu.make_async_copy(mo_v, mo_hbm, vosem).start()

    pltpu.make_async_copy(
        x_hbm.at[:, pl.ds(jb * _BLK, _BLK)], xb_in, bsem).wait()
    bcol = jb * _BLK + jax.lax.broadcasted_iota(jnp.int32, (1, _BLK), 1)
    xb_out[...] = jnp.where(bcol < ml, xb_in[...], 0.0)
    pltpu.make_async_copy(
        xb_out, xo_hbm.at[:, pl.ds(jb * _BLK, _BLK)], bsem).start()

    for d in range(nblk):
        @pl.when(d < jb)
        def _wc(d=d):
            pltpu.make_async_copy(
                x_hbm.at[:, pl.ds(d * _BLK, _BLK)],
                xo_hbm.at[:, pl.ds(d * _BLK, _BLK)], csem).wait()

        @pl.when(d > jb)
        def _wz(d=d):
            pltpu.make_async_copy(
                zbuf, xo_hbm.at[:, pl.ds(d * _BLK, _BLK)], zsem).wait()

    pltpu.make_async_copy(
        xb_out, xo_hbm.at[:, pl.ds(jb * _BLK, _BLK)], bsem).wait()
    pltpu.make_async_copy(vo_v, vo_hbm, vosem).wait()
    pltpu.make_async_copy(mo_v, mo_hbm, vosem).wait()


def kernel(x, v, mask):
    B, C, P = x.shape
    CV = v.shape[1]
    R = B * C
    nblk = P // _BLK
    xr = x.reshape(R, P)
    vr = v.reshape(B * CV, P)
    mr = mask.reshape(B, P)

    body = functools.partial(_trim_body, nblk)
    hbm = pl.BlockSpec(memory_space=pltpu.MemorySpace.HBM)

    mo, vo, xo = pl.pallas_call(
        body,
        in_specs=[hbm, hbm, hbm],
        out_specs=[hbm, hbm, hbm],
        out_shape=[
            jax.ShapeDtypeStruct((B, P), jnp.int32),
            jax.ShapeDtypeStruct((B * CV, P), jnp.float32),
            jax.ShapeDtypeStruct((R, P), jnp.float32),
        ],
        scratch_shapes=[
            pltpu.VMEM((B, P), jnp.int32),
            pltpu.VMEM((B * CV, P), jnp.float32),
            pltpu.VMEM((B, P), jnp.int32),
            pltpu.VMEM((B * CV, P), jnp.float32),
            pltpu.VMEM((R, _BLK), jnp.float32),
            pltpu.VMEM((R, _BLK), jnp.float32),
            pltpu.VMEM((R, _BLK), jnp.float32),
            pltpu.SemaphoreType.DMA,
            pltpu.SemaphoreType.DMA,
            pltpu.SemaphoreType.DMA,
            pltpu.SemaphoreType.DMA,
            pltpu.SemaphoreType.DMA,
            pltpu.SemaphoreType.DMA,
        ],
    )(mr, vr, xr)
    return (xo.reshape(B, C, P), vo.reshape(B, CV, P), mo.reshape(B, 1, P))


# speedup vs baseline: 19.4484x; 19.4484x over previous
import functools

import jax
import jax.numpy as jnp
from jax.experimental import pallas as pl
from jax.experimental.pallas import tpu as pltpu


_BLK = 512
_NIN = 4
_NOUT = 3


def _trim_body(nblk, m_hbm, v_hbm, x_hbm, mo_hbm, vo_hbm, xo_hbm,
               ml_s, m_v, v_v, mo_v, vo_v, xin, xout, zbuf,
               msem, vsem, vosem, in_sem, out_sem, zsem):
    i = pl.program_id(0)

    @pl.when(i == 0)
    def _prologue():
        pltpu.make_async_copy(m_hbm, m_v, msem).start()
        for b in (0, 1, 2):
            pltpu.make_async_copy(
                x_hbm.at[:, pl.ds(b * _BLK, _BLK)], xin.at[b],
                in_sem.at[b]).start()
        pltpu.make_async_copy(v_hbm, v_v, vsem).start()
        pltpu.make_async_copy(m_hbm, m_v, msem).wait()
        counts = jnp.sum((m_v[...] != 0).astype(jnp.int32), axis=1)
        ml0 = jnp.maximum(jnp.max(counts), 1)
        ml_s[0] = ml0

        zbuf[...] = jnp.zeros_like(zbuf)

    ml = ml_s[0]
    jlast = (ml - 1) // _BLK

    @pl.when(i == 1)
    def _small_tensors():
        col = jax.lax.broadcasted_iota(jnp.int32, (1, v_v.shape[1]), 1)
        keep = col < ml
        pltpu.make_async_copy(v_hbm, v_v, vsem).wait()
        vo_v[...] = jnp.where(keep, v_v[...], 0.0)
        mo_v[...] = jnp.where(keep & (m_v[...] != 0), 1, 0).astype(jnp.int32)
        pltpu.make_async_copy(vo_v, vo_hbm, vosem).start()
        pltpu.make_async_copy(mo_v, mo_hbm, vosem).start()

    oslot = jax.lax.rem(i, _NOUT)
    prev = i - _NOUT

    @pl.when((prev >= 0) & (prev <= jlast))
    def _wait_prev_out():
        pltpu.make_async_copy(
            xout.at[oslot], xo_hbm.at[:, pl.ds(prev * _BLK, _BLK)],
            out_sem.at[oslot]).wait()

    nxt = i + 3

    @pl.when((nxt < nblk) & (nxt <= jlast))
    def _start_next():
        pltpu.make_async_copy(
            x_hbm.at[:, pl.ds(nxt * _BLK, _BLK)],
            xin.at[jax.lax.rem(nxt, _NIN)],
            in_sem.at[jax.lax.rem(nxt, _NIN)]).start()

    bz = jlast + 1 + i

    @pl.when(bz < nblk)
    def _zero_write():
        pltpu.make_async_copy(
            zbuf, xo_hbm.at[:, pl.ds(bz * _BLK, _BLK)], zsem).start()

    @pl.when((i <= jlast) | (i <= 2))
    def _retire_in():
        islot0 = jax.lax.rem(i, _NIN)
        pltpu.make_async_copy(
            x_hbm.at[:, pl.ds(i * _BLK, _BLK)], xin.at[islot0],
            in_sem.at[islot0]).wait()

    @pl.when(i <= jlast)
    def _compute():
        islot = jax.lax.rem(i, _NIN)
        col = i * _BLK + jax.lax.broadcasted_iota(jnp.int32, (1, _BLK), 1)
        xout[oslot] = jnp.where(col < ml, xin[islot], 0.0)
        pltpu.make_async_copy(
            xout.at[oslot], xo_hbm.at[:, pl.ds(i * _BLK, _BLK)],
            out_sem.at[oslot]).start()

    @pl.when(i == nblk - 1)
    def _epilogue():
        for d in (nblk - 3, nblk - 2, nblk - 1):
            s = d % _NOUT

            @pl.when(d <= jlast)
            def _():
                pltpu.make_async_copy(
                    xout.at[s], xo_hbm.at[:, pl.ds(d * _BLK, _BLK)],
                    out_sem.at[s]).wait()

        for d in range(1, nblk):
            @pl.when(d > jlast)
            def _():
                pltpu.make_async_copy(
                    zbuf, xo_hbm.at[:, pl.ds(d * _BLK, _BLK)], zsem).wait()

        pltpu.make_async_copy(vo_v, vo_hbm, vosem).wait()
        pltpu.make_async_copy(mo_v, mo_hbm, vosem).wait()


def kernel(x, v, mask):
    B, C, P = x.shape
    CV = v.shape[1]
    R = B * C
    nblk = P // _BLK
    xr = x.reshape(R, P)
    vr = v.reshape(B * CV, P)
    mr = mask.reshape(B, P)

    body = functools.partial(_trim_body, nblk)
    hbm = pl.BlockSpec(memory_space=pltpu.MemorySpace.HBM)

    mo, vo, xo = pl.pallas_call(
        body,
        grid=(nblk,),
        in_specs=[hbm, hbm, hbm],
        out_specs=[hbm, hbm, hbm],
        out_shape=[
            jax.ShapeDtypeStruct((B, P), jnp.int32),
            jax.ShapeDtypeStruct((B * CV, P), jnp.float32),
            jax.ShapeDtypeStruct((R, P), jnp.float32),
        ],
        scratch_shapes=[
            pltpu.SMEM((1,), jnp.int32),
            pltpu.VMEM((B, P), jnp.int32),
            pltpu.VMEM((B * CV, P), jnp.float32),
            pltpu.VMEM((B, P), jnp.int32),
            pltpu.VMEM((B * CV, P), jnp.float32),
            pltpu.VMEM((_NIN, R, _BLK), jnp.float32),
            pltpu.VMEM((_NOUT, R, _BLK), jnp.float32),
            pltpu.VMEM((R, _BLK), jnp.float32),
            pltpu.SemaphoreType.DMA,
            pltpu.SemaphoreType.DMA,
            pltpu.SemaphoreType.DMA,
            pltpu.SemaphoreType.DMA((_NIN,)),
            pltpu.SemaphoreType.DMA((_NOUT,)),
            pltpu.SemaphoreType.DMA,
        ],
        compiler_params=pltpu.CompilerParams(
            dimension_semantics=("arbitrary",),
        ),
    )(mr, vr, xr)
    return (xo.reshape(B, C, P), vo.reshape(B, CV, P), mo.reshape(B, 1, P))


# speedup vs baseline: 19.9394x; 1.0252x over previous
import functools

import jax
import jax.numpy as jnp
from jax.experimental import pallas as pl
from jax.experimental.pallas import tpu as pltpu


_BLK = 512
_NIN = 4
_NOUT = 3


def _trim_body(nblk, m_hbm, v_hbm, x_hbm, mo_hbm, vo_hbm, xo_hbm,
               ml_s, m_v, v_v, mo_v, vo_v, xin, xout, zbuf,
               msem, vsem, vosem, in_sem, out_sem, zsem):
    i = pl.program_id(0)

    @pl.when(i == 0)
    def _prologue():
        pltpu.make_async_copy(m_hbm, m_v, msem).start()
        for b in (0, 1, 2):
            pltpu.make_async_copy(
                x_hbm.at[:, pl.ds(b * _BLK, _BLK)], xin.at[b],
                in_sem.at[b]).start()
        pltpu.make_async_copy(v_hbm, v_v, vsem).start()
        pltpu.make_async_copy(m_hbm, m_v, msem).wait()
        counts = jnp.sum((m_v[...] != 0).astype(jnp.int32), axis=1)
        ml0 = jnp.maximum(jnp.max(counts), 1)
        ml_s[0] = ml0

        zbuf[...] = jnp.zeros_like(zbuf)

    ml = ml_s[0]
    jlast = (ml - 1) // _BLK

    @pl.when(i == 1)
    def _small_tensors():
        col = jax.lax.broadcasted_iota(jnp.int32, (1, v_v.shape[1]), 1)
        keep = col < ml
        pltpu.make_async_copy(v_hbm, v_v, vsem).wait()
        vo_v[...] = jnp.where(keep, v_v[...], 0.0)
        mo_v[...] = jnp.where(keep & (m_v[...] != 0), 1, 0).astype(jnp.int32)
        pltpu.make_async_copy(vo_v, vo_hbm, vosem).start()
        pltpu.make_async_copy(mo_v, mo_hbm, vosem).start()

    oslot = jax.lax.rem(i, _NOUT)
    prev = i - _NOUT

    @pl.when((prev >= 0) & (prev <= jlast))
    def _wait_prev_out():
        pltpu.make_async_copy(
            xout.at[oslot], xo_hbm.at[:, pl.ds(prev * _BLK, _BLK)],
            out_sem.at[oslot]).wait()

    nxt = i + 3

    half = _BLK // 2
    nxt_lo_only = (nxt == jlast) & (ml <= nxt * _BLK + half)

    @pl.when((nxt < nblk) & (nxt <= jlast))
    def _start_next():
        nsl = jax.lax.rem(nxt, _NIN)

        @pl.when(nxt_lo_only)
        def _():
            pltpu.make_async_copy(
                x_hbm.at[:, pl.ds(nxt * _BLK, half)],
                xin.at[nsl, :, pl.ds(0, half)],
                in_sem.at[nsl]).start()

        @pl.when(jnp.logical_not(nxt_lo_only))
        def _():
            pltpu.make_async_copy(
                x_hbm.at[:, pl.ds(nxt * _BLK, _BLK)],
                xin.at[nsl], in_sem.at[nsl]).start()

    bz = jlast + 1 + i

    @pl.when(bz < nblk)
    def _zero_write():
        pltpu.make_async_copy(
            zbuf, xo_hbm.at[:, pl.ds(bz * _BLK, _BLK)], zsem).start()

    i_lo_only = (i >= 3) & (i == jlast) & (ml <= i * _BLK + half)

    @pl.when((i <= jlast) | (i <= 2))
    def _retire_in():
        islot0 = jax.lax.rem(i, _NIN)

        @pl.when(i_lo_only)
        def _():
            pltpu.make_async_copy(
                x_hbm.at[:, pl.ds(i * _BLK, half)],
                xin.at[islot0, :, pl.ds(0, half)],
                in_sem.at[islot0]).wait()

        @pl.when(jnp.logical_not(i_lo_only))
        def _():
            pltpu.make_async_copy(
                x_hbm.at[:, pl.ds(i * _BLK, _BLK)], xin.at[islot0],
                in_sem.at[islot0]).wait()

    @pl.when(i <= jlast)
    def _compute():
        islot = jax.lax.rem(i, _NIN)
        col = i * _BLK + jax.lax.broadcasted_iota(jnp.int32, (1, _BLK), 1)
        xout[oslot] = jnp.where(col < ml, xin[islot], 0.0)
        pltpu.make_async_copy(
            xout.at[oslot], xo_hbm.at[:, pl.ds(i * _BLK, _BLK)],
            out_sem.at[oslot]).start()

    @pl.when(i == nblk - 1)
    def _epilogue():
        for d in (nblk - 3, nblk - 2, nblk - 1):
            s = d % _NOUT

            @pl.when(d <= jlast)
            def _():
                pltpu.make_async_copy(
                    xout.at[s], xo_hbm.at[:, pl.ds(d * _BLK, _BLK)],
                    out_sem.at[s]).wait()

        for d in range(1, nblk):
            @pl.when(d > jlast)
            def _():
                pltpu.make_async_copy(
                    zbuf, xo_hbm.at[:, pl.ds(d * _BLK, _BLK)], zsem).wait()

        pltpu.make_async_copy(vo_v, vo_hbm, vosem).wait()
        pltpu.make_async_copy(mo_v, mo_hbm, vosem).wait()


def kernel(x, v, mask):
    B, C, P = x.shape
    CV = v.shape[1]
    R = B * C
    nblk = P // _BLK
    xr = x.reshape(R, P)
    vr = v.reshape(B * CV, P)
    mr = mask.reshape(B, P)

    body = functools.partial(_trim_body, nblk)
    hbm = pl.BlockSpec(memory_space=pltpu.MemorySpace.HBM)

    mo, vo, xo = pl.pallas_call(
        body,
        grid=(nblk,),
        in_specs=[hbm, hbm, hbm],
        out_specs=[hbm, hbm, hbm],
        out_shape=[
            jax.ShapeDtypeStruct((B, P), jnp.int32),
            jax.ShapeDtypeStruct((B * CV, P), jnp.float32),
            jax.ShapeDtypeStruct((R, P), jnp.float32),
        ],
        scratch_shapes=[
            pltpu.SMEM((1,), jnp.int32),
            pltpu.VMEM((B, P), jnp.int32),
            pltpu.VMEM((B * CV, P), jnp.float32),
            pltpu.VMEM((B, P), jnp.int32),
            pltpu.VMEM((B * CV, P), jnp.float32),
            pltpu.VMEM((_NIN, R, _BLK), jnp.float32),
            pltpu.VMEM((_NOUT, R, _BLK), jnp.float32),
            pltpu.VMEM((R, _BLK), jnp.float32),
            pltpu.SemaphoreType.DMA,
            pltpu.SemaphoreType.DMA,
            pltpu.SemaphoreType.DMA,
            pltpu.SemaphoreType.DMA((_NIN,)),
            pltpu.SemaphoreType.DMA((_NOUT,)),
            pltpu.SemaphoreType.DMA,
        ],
        compiler_params=pltpu.CompilerParams(
            dimension_semantics=("arbitrary",),
        ),
    )(mr, vr, xr)
    return (xo.reshape(B, C, P), vo.reshape(B, CV, P), mo.reshape(B, 1, P))
